# baseline (speedup 1.0000x reference)
import functools

import jax
import jax.numpy as jnp
from jax import lax
from jax.experimental import pallas as pl
from jax.experimental.pallas import tpu as pltpu
from jax.experimental.pallas import tpu_sc as plsc

B = 128
S = 8
V = 100000
ROWS = B * S
NC, NS, L = 2, 16, 16
NW = NC * NS
RPW = ROWS // NW
NCHUNK = 10
CH = V // NCHUNK
VREGS = CH // L

_BIG = 2**30


def _argmax_sc_body(logits_hbm, maxv_hbm, idxv_hbm, buf, maxb, idxb):
    wid = lax.axis_index("s") * NC + lax.axis_index("c")
    row0 = wid * RPW
    iota = lax.broadcasted_iota(jnp.int32, (L,), 0)

    def row_body(rr, unused):
        base = (row0 + rr) * V
        rm = jnp.full((L,), -3.4e38, jnp.float32)
        ri = jnp.zeros((L,), jnp.int32)
        for k in range(NCHUNK):
            pltpu.sync_copy(logits_hbm.at[pl.ds(base + k * CH, CH)], buf)

            def vbody(i, c):
                rm, ri, iv = c
                v = buf[pl.ds(i * L, L)]
                m = v > rm
                rm = jnp.maximum(rm, v)
                ri = jnp.where(m, iv, ri)
                return rm, ri, iv + L

            rm, ri, _ = lax.fori_loop(0, VREGS, vbody,
                                      (rm, ri, iota + k * CH))
        maxb[pl.ds(rr * L, L)] = rm
        idxb[pl.ds(rr * L, L)] = ri
        return unused

    lax.fori_loop(0, RPW, row_body, 0)
    pltpu.sync_copy(maxb, maxv_hbm.at[pl.ds(row0 * L, RPW * L)])
    pltpu.sync_copy(idxb, idxv_hbm.at[pl.ds(row0 * L, RPW * L)])


_argmax_sc = functools.partial(
    pl.kernel,
    out_type=(
        jax.ShapeDtypeStruct((ROWS * L,), jnp.float32),
        jax.ShapeDtypeStruct((ROWS * L,), jnp.int32),
    ),
    mesh=plsc.VectorSubcoreMesh(core_axis_name="c", subcore_axis_name="s",
                                num_cores=NC, num_subcores=NS),
    scratch_types=[
        pltpu.VMEM((CH,), jnp.float32),
        pltpu.VMEM((RPW * L,), jnp.float32),
        pltpu.VMEM((RPW * L,), jnp.int32),
    ],
)(_argmax_sc_body)


def _finish_body(maxv_ref, idxv_ref, draft_ref, bonus_ref,
                 out_ref, nrej_ref, last_ref):
    maxv = maxv_ref[...]
    idxv = idxv_ref[...]
    dr = draft_ref[...]
    bo = bonus_ref[...]
    vmax = jnp.max(maxv, axis=2, keepdims=True)
    tok = jnp.min(jnp.where(maxv == vmax, idxv, _BIG), axis=2, keepdims=True)
    io = lax.broadcasted_iota(jnp.int32, (B, S, 1), 1)
    m = tok == dr
    n = jnp.min(jnp.where(m, S, io), axis=1, keepdims=True)
    out_ref[:, :S] = jnp.where(io <= n, tok, -1)
    out_ref[:, S:] = jnp.where(n == S, bo, -1)
    nrej_ref[...] = S - n
    lastt = jnp.sum(jnp.where(io == n, tok, 0), axis=1, keepdims=True)
    last_ref[...] = jnp.where(n == S, bo, lastt)


_finish_tc = pl.pallas_call(
    _finish_body,
    out_shape=[
        jax.ShapeDtypeStruct((B, S + 1, 1), jnp.int32),
        jax.ShapeDtypeStruct((B, 1, 1), jnp.int32),
        jax.ShapeDtypeStruct((B, 1, 1), jnp.int32),
    ],
)


def kernel(target_logits, draft_token_ids, bonus_token_ids):
    flat = target_logits.reshape(ROWS * V)
    maxv, idxv = _argmax_sc(flat)
    out, nrej, last = _finish_tc(
        maxv.reshape(B, S, L), idxv.reshape(B, S, L),
        draft_token_ids.reshape(B, S, 1), bonus_token_ids.reshape(B, 1, 1))
    return out.reshape(B, S + 1), nrej.reshape(B), last.reshape(B)


# speedup vs baseline: 1.9745x; 1.9745x over previous
import functools

import jax
import jax.numpy as jnp
from jax import lax
from jax.experimental import pallas as pl
from jax.experimental.pallas import tpu as pltpu
from jax.experimental.pallas import tpu_sc as plsc

B = 128
S = 8
V = 100000
ROWS = B * S
NC, NS, L = 2, 16, 16
NW = NC * NS
RPW = ROWS // NW
NCHUNK = 5
CH = V // NCHUNK
A = 10
ITERS = CH // (A * L)

_BIG = 2**30


def _argmax_sc_body(logits_hbm, maxv_hbm, idxv_hbm,
                    bufa, bufb, maxb, idxb, sema, semb):
    wid = lax.axis_index("s") * NC + lax.axis_index("c")
    row0 = wid * RPW
    iota = lax.broadcasted_iota(jnp.int32, (L,), 0)
    bufs = (bufa, bufb)
    sems = (sema, semb)

    def start(r, k, which):
        pltpu.async_copy(logits_hbm.at[pl.ds(r * V + k * CH, CH)],
                         bufs[which], sems[which])

    def wait(r, k, which):
        pltpu.make_async_copy(logits_hbm.at[pl.ds(r * V + k * CH, CH)],
                              bufs[which], sems[which]).wait()

    def run_chunk(buf, it0, rms, ris):
        def ibody(i, carry):
            rms, ris = carry
            itv = jnp.full((L,), it0 + i, jnp.int32)
            base = i * (A * L)
            nm, ni = [], []
            for j in range(A):
                v = buf[pl.ds(base + j * L, L)]
                m = v > rms[j]
                nm.append(jnp.maximum(rms[j], v))
                ni.append(jnp.where(m, itv, ris[j]))
            return tuple(nm), tuple(ni)

        return lax.fori_loop(0, ITERS, ibody, (rms, ris))

    def finalize(rr, rms, ris):
        bm = rms[0]
        bi = ris[0] * (A * L) + iota
        for j in range(1, A):
            b = rms[j]
            ib = ris[j] * (A * L) + (j * L) + iota
            take = (b > bm) | ((b == bm) & (ib < bi))
            bm = jnp.where(take, b, bm)
            bi = jnp.where(take, ib, bi)
        maxb[pl.ds((rr - row0) * L, L)] = bm
        idxb[pl.ds((rr - row0) * L, L)] = bi

    start(row0, 0, 0)

    def pair_body(t, unused):
        r_even = row0 + 2 * t
        rms = ris = None
        for u in range(2 * NCHUNK):
            rr = r_even + (u // NCHUNK)
            k = u % NCHUNK
            if u < 2 * NCHUNK - 1:
                rn = r_even + ((u + 1) // NCHUNK)
                kn = (u + 1) % NCHUNK
            else:
                rn = jnp.minimum(r_even + 2, row0 + RPW - 1)
                kn = 0
            start(rn, kn, (u + 1) % 2)
            wait(rr, k, u % 2)
            if k == 0:
                rms = tuple(jnp.full((L,), -3.4e38, jnp.float32)
                            for _ in range(A))
                ris = tuple(jnp.zeros((L,), jnp.int32) for _ in range(A))
            rms, ris = run_chunk(bufs[u % 2], k * ITERS, rms, ris)
            if k == NCHUNK - 1:
                finalize(rr, rms, ris)
        return unused

    lax.fori_loop(0, RPW // 2, pair_body, 0)
    wait(row0 + RPW - 1, 0, 0)
    pltpu.sync_copy(maxb, maxv_hbm.at[pl.ds(row0 * L, RPW * L)])
    pltpu.sync_copy(idxb, idxv_hbm.at[pl.ds(row0 * L, RPW * L)])


_argmax_sc = functools.partial(
    pl.kernel,
    out_type=(
        jax.ShapeDtypeStruct((ROWS * L,), jnp.float32),
        jax.ShapeDtypeStruct((ROWS * L,), jnp.int32),
    ),
    mesh=plsc.VectorSubcoreMesh(core_axis_name="c", subcore_axis_name="s",
                                num_cores=NC, num_subcores=NS),
    scratch_types=[
        pltpu.VMEM((CH,), jnp.float32),
        pltpu.VMEM((CH,), jnp.float32),
        pltpu.VMEM((RPW * L,), jnp.float32),
        pltpu.VMEM((RPW * L,), jnp.int32),
        pltpu.SemaphoreType.DMA,
        pltpu.SemaphoreType.DMA,
    ],
)(_argmax_sc_body)


def _fold_body(maxv_ref, idxv_ref, tok_ref):
    maxv = maxv_ref[...]
    idxv = idxv_ref[...]
    vmax = jnp.max(maxv, axis=1, keepdims=True)
    tok_ref[...] = jnp.min(jnp.where(maxv == vmax, idxv, _BIG),
                           axis=1, keepdims=True)


_fold_tc = pl.pallas_call(
    _fold_body,
    out_shape=jax.ShapeDtypeStruct((ROWS, 1), jnp.int32),
)


def _finish_body(tok_ref, draft_ref, bonus_ref, out_ref, nrej_ref, last_ref):
    tok = tok_ref[...]
    dr = draft_ref[...]
    bo = bonus_ref[...]
    io = lax.broadcasted_iota(jnp.int32, (B, S), 1)
    m = tok == dr
    n = jnp.min(jnp.where(m, S, io), axis=1, keepdims=True)
    out_ref[:, :S] = jnp.where(io <= n, tok, -1)
    out_ref[:, S:] = jnp.where(n == S, bo, -1)
    nrej_ref[...] = S - n
    lastt = jnp.sum(jnp.where(io == n, tok, 0), axis=1, keepdims=True)
    last_ref[...] = jnp.where(n == S, bo, lastt)


_finish_tc = pl.pallas_call(
    _finish_body,
    out_shape=[
        jax.ShapeDtypeStruct((B, S + 1), jnp.int32),
        jax.ShapeDtypeStruct((B, 1), jnp.int32),
        jax.ShapeDtypeStruct((B, 1), jnp.int32),
    ],
)


def kernel(target_logits, draft_token_ids, bonus_token_ids):
    flat = target_logits.reshape(ROWS * V)
    maxv, idxv = _argmax_sc(flat)
    tok = _fold_tc(maxv.reshape(ROWS, L), idxv.reshape(ROWS, L))
    out, nrej, last = _finish_tc(tok.reshape(B, S),
                                 draft_token_ids, bonus_token_ids)
    return out, nrej.reshape(B), last.reshape(B)
